# baseline (speedup 1.0000x reference)
import functools

import jax
import jax.numpy as jnp
from jax import lax
from jax.experimental import pallas as pl
from jax.experimental.pallas import tpu as pltpu
from jax.experimental.pallas import tpu_sc as plsc

_CHUNK = 512


def _sc_gather(nbr_flat, table, k2, n_pad, cin):
    info = plsc.get_sparse_core_info()
    nc, ns = info.num_cores, info.num_subcores
    nw = nc * ns
    per_w = n_pad // nw
    assert per_w * nw == n_pad
    chunk = _CHUNK
    chunks = per_w // chunk
    assert chunks * chunk == per_w and chunk % 16 == 0
    jc = k2 * cin

    mesh = plsc.VectorSubcoreMesh(core_axis_name="c", subcore_axis_name="s")

    @functools.partial(
        pl.kernel,
        out_type=jax.ShapeDtypeStruct((jc, n_pad), jnp.float32),
        mesh=mesh,
        scratch_types=[
            pltpu.VMEM((chunk * k2,), jnp.int32),
            pltpu.VMEM((chunk * k2, cin), jnp.float32),
            pltpu.VMEM((jc, chunk), jnp.float32),
            pltpu.SemaphoreType.DMA,
        ],
        compiler_params=pltpu.CompilerParams(
            use_tc_tiling_on_sc=False, needs_layout_passes=False
        ),
    )
    def gather_kernel(nbr_hbm, tab_hbm, g_hbm, idx_v, gbuf, gt, sem_g):
        wid = lax.axis_index("s") * nc + lax.axis_index("c")
        base0 = wid * per_w
        iota9 = lax.iota(jnp.int32, 16) * k2
        cvecs = [jnp.full((16,), c, jnp.int32) for c in range(cin)]

        def body(ci, carry):
            base = base0 + ci * chunk
            pltpu.sync_copy(
                nbr_hbm.at[pl.ds(base * k2, chunk * k2)], idx_v
            )
            pltpu.async_copy(tab_hbm.at[idx_v], gbuf, sem_g).wait()

            def tbody(g, carry2):
                pbase = g * 16
                ro = pbase * k2
                for k in range(k2):
                    ridx = iota9 + (ro + k)
                    for c in range(cin):
                        v = plsc.load_gather(gbuf, [ridx, cvecs[c]])
                        gt[k * cin + c, pl.ds(pbase, 16)] = v
                return carry2

            lax.fori_loop(0, chunk // 16, tbody, 0)
            pltpu.sync_copy(gt, g_hbm.at[:, pl.ds(base, chunk)])
            return carry

        lax.fori_loop(0, chunks, body, 0)

    return gather_kernel(nbr_flat, table)


def _tc_apply(g, w32, ld2, bias2, n, jc, nstencils, cout):
    bn = 1280
    nb = n // bn
    assert nb * bn == n

    def body(g_ref, w_ref, ld_ref, b_ref, o_ref):
        acc = jnp.dot(w_ref[...], g_ref[...], preferred_element_type=jnp.float32)
        ld = ld_ref[...]
        out = jnp.zeros((cout, bn), jnp.float32)
        for s in range(nstencils):
            out = out + jnp.where(
                ld == s, acc[s * cout : (s + 1) * cout, :], 0.0
            )
        o_ref[...] = out + b_ref[...]

    return pl.pallas_call(
        body,
        grid=(nb,),
        in_specs=[
            pl.BlockSpec((jc, bn), lambda i: (0, i)),
            pl.BlockSpec((nstencils * cout, jc), lambda i: (0, 0)),
            pl.BlockSpec((1, bn), lambda i: (0, i)),
            pl.BlockSpec((cout, 1), lambda i: (0, 0)),
        ],
        out_specs=pl.BlockSpec((cout, bn), lambda i: (0, i)),
        out_shape=jax.ShapeDtypeStruct((cout, n), jnp.float32),
        compiler_params=pltpu.CompilerParams(
            dimension_semantics=("arbitrary",),
        ),
    )(g, w32, ld2, bias2)


def kernel(intensities, weight, bias, neighbors, level_deltas):
    b, cin, n = intensities.shape
    cout, _, nstencils, kh, kw = weight.shape
    k2 = kh * kw
    jc = k2 * cin

    n_pad = 819200

    table = intensities[0].T
    nbr_flat = jnp.pad(
        neighbors.astype(jnp.int32).reshape(-1), (0, (n_pad - n) * k2)
    )
    ld2 = level_deltas.astype(jnp.int32).reshape(1, n)
    w32 = jnp.transpose(weight, (2, 0, 3, 4, 1)).reshape(
        nstencils * cout, jc
    )
    bias2 = bias.reshape(cout, 1)

    g = _sc_gather(nbr_flat, table, k2, n_pad, cin)
    out = _tc_apply(g, w32, ld2, bias2, n, jc, nstencils, cout)
    return out.reshape(b, cout, n)


# speedup vs baseline: 4.7021x; 4.7021x over previous
import functools

import jax
import jax.numpy as jnp
from jax import lax
from jax.experimental import pallas as pl
from jax.experimental.pallas import tpu as pltpu
from jax.experimental.pallas import tpu_sc as plsc

_CHUNK = 512


def _sc_gather(nbr_flat, table, k2, n, cin):
    info = plsc.get_sparse_core_info()
    nc, ns = info.num_cores, info.num_subcores
    nw = nc * ns
    chunk = _CHUNK
    nchunk = (n + chunk - 1) // chunk
    per_w = (nchunk + nw - 1) // nw
    jc = k2 * cin
    tpc = chunk // 128
    last_base = n - chunk
    ng = chunk // 16

    mesh = plsc.VectorSubcoreMesh(core_axis_name="c", subcore_axis_name="s")

    @functools.partial(
        pl.kernel,
        out_type=jax.ShapeDtypeStruct((jc // 8, n // 128, 8, 128), jnp.float32),
        mesh=mesh,
        scratch_types=[
            pltpu.VMEM((chunk * k2,), jnp.int32),
            pltpu.VMEM((chunk * k2,), jnp.int32),
            pltpu.VMEM((chunk * k2, cin), jnp.float32),
            pltpu.VMEM((chunk * k2, cin), jnp.float32),
            pltpu.VMEM((jc // 8, tpc, 8, 128), jnp.float32),
            pltpu.SemaphoreType.DMA,
            pltpu.SemaphoreType.DMA,
        ],
        compiler_params=pltpu.CompilerParams(
            use_tc_tiling_on_sc=False, needs_layout_passes=False
        ),
    )
    def gather_kernel(
        nbr_hbm, tab_hbm, g_hbm, idx0, idx1, gb0, gb1, gt4, sem0, sem1
    ):
        wid = lax.axis_index("s") * nc + lax.axis_index("c")
        iota9 = lax.iota(jnp.int32, 16) * k2
        cvecs = [jnp.full((16,), c, jnp.int32) for c in range(cin)]

        def cbase(q):
            return jnp.minimum((wid + q * nw) * chunk, last_base)

        def fire(q, idxv, gb, sem):
            b = cbase(q)
            pltpu.sync_copy(nbr_hbm.at[pl.ds(b * k2, chunk * k2)], idxv)
            pltpu.async_copy(tab_hbm.at[idxv], gb, sem)

        def wait(idxv, gb, sem):
            pltpu.make_async_copy(tab_hbm.at[idxv], gb, sem).wait()

        def transpose_store(q, gb):
            def tbody(g, c2):
                pbase = g * 16
                ro = pbase * k2
                t = g // 8
                lo = (g % 8) * 16
                for k in range(k2):
                    ridx = iota9 + (ro + k)
                    for c in range(cin):
                        v = plsc.load_gather(gb, [ridx, cvecs[c]])
                        j = k * cin + c
                        gt4[j // 8, t, j % 8, pl.ds(lo, 16)] = v
                return c2

            lax.fori_loop(0, ng, tbody, 0)
            b = cbase(q)
            pltpu.sync_copy(
                gt4, g_hbm.at[:, pl.ds(b // 128, tpc), :, :]
            )

        fire(0, idx0, gb0, sem0)

        def body(p, carry):
            q0 = 2 * p
            fire(q0 + 1, idx1, gb1, sem1)
            wait(idx0, gb0, sem0)
            transpose_store(q0, gb0)
            fire(q0 + 2, idx0, gb0, sem0)
            wait(idx1, gb1, sem1)
            transpose_store(q0 + 1, gb1)
            return carry

        lax.fori_loop(0, (per_w - 1) // 2, body, 0)
        wait(idx0, gb0, sem0)
        transpose_store(per_w - 1, gb0)

    return gather_kernel(nbr_flat, table)


def _tc_apply(g4, w32, ld2, bias2, n, jc, nstencils, cout):
    bt = 8
    bn = bt * 128
    nb = (n + bn - 1) // bn

    def body(g_ref, w_ref, ld_ref, b_ref, o_ref):
        for t in range(bt):
            g72 = g_ref[:, t].reshape(jc, 128)
            acc = jnp.dot(w_ref[...], g72, preferred_element_type=jnp.float32)
            ld = ld_ref[:, t * 128 : (t + 1) * 128]
            out = jnp.zeros((cout, 128), jnp.float32)
            for s in range(nstencils):
                out = out + jnp.where(
                    ld == s, acc[s * cout : (s + 1) * cout, :], 0.0
                )
            o_ref[:, t * 128 : (t + 1) * 128] = out + b_ref[...]

    return pl.pallas_call(
        body,
        grid=(nb,),
        in_specs=[
            pl.BlockSpec((jc // 8, bt, 8, 128), lambda i: (0, i, 0, 0)),
            pl.BlockSpec((nstencils * cout, jc), lambda i: (0, 0)),
            pl.BlockSpec((1, bn), lambda i: (0, i)),
            pl.BlockSpec((cout, 1), lambda i: (0, 0)),
        ],
        out_specs=pl.BlockSpec((cout, bn), lambda i: (0, i)),
        out_shape=jax.ShapeDtypeStruct((cout, n), jnp.float32),
        compiler_params=pltpu.CompilerParams(
            dimension_semantics=("arbitrary",),
        ),
    )(g4, w32, ld2, bias2)


def kernel(intensities, weight, bias, neighbors, level_deltas):
    b, cin, n = intensities.shape
    cout, _, nstencils, kh, kw = weight.shape
    k2 = kh * kw
    jc = k2 * cin

    table = intensities[0].T
    nbr_flat = neighbors.astype(jnp.int32).reshape(-1)
    ld2 = level_deltas.astype(jnp.int32).reshape(1, n)
    w32 = jnp.transpose(weight, (2, 0, 3, 4, 1)).reshape(nstencils * cout, jc)
    bias2 = bias.reshape(cout, 1)

    g4 = _sc_gather(nbr_flat, table, k2, n, cin)
    out = _tc_apply(g4, w32, ld2, bias2, n, jc, nstencils, cout)
    return out.reshape(b, cout, n)


# speedup vs baseline: 5.5439x; 1.1790x over previous
import functools

import jax
import jax.numpy as jnp
from jax import lax
from jax.experimental import pallas as pl
from jax.experimental.pallas import tpu as pltpu
from jax.experimental.pallas import tpu_sc as plsc

_CHUNK = 512


def _sc_gather(nbr_flat, table, k2, n, cin, start, count):
    info = plsc.get_sparse_core_info()
    nc, ns = info.num_cores, info.num_subcores
    nw = nc * ns
    chunk = _CHUNK
    nchunk = (count + chunk - 1) // chunk
    per_w = (nchunk + nw - 1) // nw
    jc = k2 * cin
    tpc = chunk // 128
    last_base = count - chunk
    ng = chunk // 16

    mesh = plsc.VectorSubcoreMesh(core_axis_name="c", subcore_axis_name="s")

    @functools.partial(
        pl.kernel,
        out_type=jax.ShapeDtypeStruct(
            (jc // 8, count // 128, 8, 128), jnp.float32
        ),
        mesh=mesh,
        scratch_types=[
            pltpu.VMEM((chunk * k2,), jnp.int32),
            pltpu.VMEM((chunk * k2,), jnp.int32),
            pltpu.VMEM((chunk * k2, cin), jnp.float32),
            pltpu.VMEM((chunk * k2, cin), jnp.float32),
            pltpu.VMEM((jc // 8, tpc, 8, 128), jnp.float32),
            pltpu.SemaphoreType.DMA,
            pltpu.SemaphoreType.DMA,
        ],
        compiler_params=pltpu.CompilerParams(
            use_tc_tiling_on_sc=False, needs_layout_passes=False
        ),
    )
    def gather_kernel(
        nbr_hbm, tab_hbm, g_hbm, idx0, idx1, gb0, gb1, gt4, sem0, sem1
    ):
        wid = lax.axis_index("s") * nc + lax.axis_index("c")
        iota9 = lax.iota(jnp.int32, 16) * k2
        cvecs = [jnp.full((16,), c, jnp.int32) for c in range(cin)]

        def cbase(q):
            return jnp.minimum((wid + q * nw) * chunk, last_base)

        def fire(q, idxv, gb, sem):
            b = cbase(q) + start
            pltpu.sync_copy(nbr_hbm.at[pl.ds(b * k2, chunk * k2)], idxv)
            pltpu.async_copy(tab_hbm.at[idxv], gb, sem)

        def wait(idxv, gb, sem):
            pltpu.make_async_copy(tab_hbm.at[idxv], gb, sem).wait()

        def transpose_store(q, gb):
            def tbody(g, c2):
                pbase = g * 16
                ro = pbase * k2
                t = g // 8
                lo = (g % 8) * 16
                for k in range(k2):
                    ridx = iota9 + (ro + k)
                    for c in range(cin):
                        v = plsc.load_gather(gb, [ridx, cvecs[c]])
                        j = k * cin + c
                        gt4[j // 8, t, j % 8, pl.ds(lo, 16)] = v
                return c2

            lax.fori_loop(0, ng, tbody, 0)
            b = cbase(q)
            pltpu.sync_copy(
                gt4, g_hbm.at[:, pl.ds(b // 128, tpc), :, :]
            )

        fire(0, idx0, gb0, sem0)

        def body(p, carry):
            q0 = 2 * p
            fire(q0 + 1, idx1, gb1, sem1)
            wait(idx0, gb0, sem0)
            transpose_store(q0, gb0)
            fire(q0 + 2, idx0, gb0, sem0)
            wait(idx1, gb1, sem1)
            transpose_store(q0 + 1, gb1)
            return carry

        lax.fori_loop(0, (per_w - 1) // 2, body, 0)
        wait(idx0, gb0, sem0)
        transpose_store(per_w - 1, gb0)

    assert per_w % 2 == 1
    return gather_kernel(nbr_flat, table)


def _tc_apply(g4, w32, ld2, bias2, n, jc, nstencils, cout):
    bt = 16
    bn = bt * 128
    nb = (n + bn - 1) // bn

    def body(g_ref, w_ref, ld_ref, b_ref, o_ref):
        for t in range(bt):
            g72 = g_ref[:, t].reshape(jc, 128)
            acc = jnp.dot(w_ref[...], g72, preferred_element_type=jnp.float32)
            ld = ld_ref[:, t * 128 : (t + 1) * 128]
            out = jnp.zeros((cout, 128), jnp.float32)
            for s in range(nstencils):
                out = out + jnp.where(
                    ld == s, acc[s * cout : (s + 1) * cout, :], 0.0
                )
            o_ref[:, t * 128 : (t + 1) * 128] = out + b_ref[...]

    return pl.pallas_call(
        body,
        grid=(nb,),
        in_specs=[
            pl.BlockSpec((jc // 8, bt, 8, 128), lambda i: (0, i, 0, 0)),
            pl.BlockSpec((nstencils * cout, jc), lambda i: (0, 0)),
            pl.BlockSpec((1, bn), lambda i: (0, i)),
            pl.BlockSpec((cout, 1), lambda i: (0, 0)),
        ],
        out_specs=pl.BlockSpec((cout, bn), lambda i: (0, i)),
        out_shape=jax.ShapeDtypeStruct((cout, n), jnp.float32),
        compiler_params=pltpu.CompilerParams(
            dimension_semantics=("arbitrary",),
        ),
    )(g4, w32, ld2, bias2)


def kernel(intensities, weight, bias, neighbors, level_deltas):
    b, cin, n = intensities.shape
    cout, _, nstencils, kh, kw = weight.shape
    k2 = kh * kw
    jc = k2 * cin

    table = intensities[0].T
    nbr_flat = neighbors.astype(jnp.int32).reshape(-1)
    ld2 = level_deltas.astype(jnp.int32).reshape(1, n)
    w32 = jnp.transpose(weight, (2, 0, 3, 4, 1)).reshape(nstencils * cout, jc)
    bias2 = bias.reshape(cout, 1)

    half = n // 2
    outs = []
    for h in range(2):
        g4 = _sc_gather(nbr_flat, table, k2, n, cin, h * half, half)
        ld_h = lax.slice(ld2, (0, h * half), (1, (h + 1) * half))
        outs.append(
            _tc_apply(g4, w32, ld_h, bias2, half, jc, nstencils, cout)
        )
    return jnp.concatenate(outs, axis=1).reshape(b, cout, n)
